# baseline (speedup 1.0000x reference)
import functools

import jax
import jax.numpy as jnp
from jax import lax
from jax.experimental import pallas as pl
from jax.experimental.pallas import tpu as pltpu
from jax.experimental.pallas import tpu_sc as plsc

N_NODES = 10000
N_EDGES = 320000
D_IN = 128
D_HID = 128
D_OUT = 64

NC = 2
NS = 16
NW = NC * NS

N_PAD = 10240
E_PAD = 327680
EW = E_PAD // NW
CHUNK = 128
NCHUNK = EW // CHUNK
RPS = N_PAD // NS

PAD_SRC = 10000
PAD_DST = 10016

_mesh = plsc.VectorSubcoreMesh(
    core_axis_name="c", subcore_axis_name="s", num_cores=NC, num_subcores=NS
)


def _deg_body(dst_hbm, zeros_hbm, out_hbm, dst_v, ones_v, acc_sh):
    cid = lax.axis_index("c")
    sid = lax.axis_index("s")
    wid = cid * NS + sid
    pltpu.sync_copy(
        zeros_hbm.at[pl.ds(sid * RPS, RPS)], acc_sh.at[pl.ds(sid * RPS, RPS)]
    )
    for j in range(CHUNK // 16):
        ones_v[pl.ds(j * 16, 16)] = jnp.full((16,), 1.0, jnp.float32)
    plsc.subcore_barrier()

    base = wid * EW

    def step(i, carry):
        off = base + i * CHUNK
        pltpu.sync_copy(dst_hbm.at[pl.ds(off, CHUNK)], dst_v)
        pltpu.sync_copy(ones_v, acc_sh.at[dst_v], add=True)
        return carry

    lax.fori_loop(0, NCHUNK, step, 0)
    plsc.subcore_barrier()
    pltpu.sync_copy(
        acc_sh.at[pl.ds(sid * RPS, RPS)], out_hbm.at[cid, pl.ds(sid * RPS, RPS)]
    )


_deg_kernel = functools.partial(
    pl.kernel,
    out_type=jax.ShapeDtypeStruct((NC, N_PAD), jnp.float32),
    mesh=_mesh,
    scratch_types=[
        pltpu.VMEM((CHUNK,), jnp.int32),
        pltpu.VMEM((CHUNK,), jnp.float32),
        pltpu.VMEM_SHARED((N_PAD,), jnp.float32),
    ],
)(_deg_body)


def _conv_body(R, ch, h_hbm, sd_hbm, zeros_hbm, out_hbm,
               idx_v, rows_v, acc_sh, sem_i, sem_g, sem_s):
    K = R // 2
    NI = 2 * R
    M = EW // ch
    cid = lax.axis_index("c")
    sid = lax.axis_index("s")
    wid = cid * NS + sid
    pltpu.sync_copy(
        zeros_hbm.at[pl.ds(sid * RPS, RPS)], acc_sh.at[pl.ds(sid * RPS, RPS)]
    )
    cbase = wid * M

    def start_idx(islot, chunk):
        pltpu.async_copy(sd_hbm.at[cbase + chunk], idx_v.at[islot], sem_i.at[islot])

    def wait_idx(islot):
        pltpu.make_async_copy(
            sd_hbm.at[cbase], idx_v.at[islot], sem_i.at[islot]
        ).wait()

    def start_gather(b, islot):
        pltpu.async_copy(h_hbm.at[idx_v.at[islot, 0]], rows_v.at[b], sem_g.at[b])

    def wait_gather(b, islot):
        pltpu.make_async_copy(
            h_hbm.at[idx_v.at[islot, 0]], rows_v.at[b], sem_g.at[b]
        ).wait()

    def start_scatter(b, islot):
        pltpu.async_copy(
            rows_v.at[b], acc_sh.at[idx_v.at[islot, 1]], sem_s.at[b], add=True
        )

    def wait_scatter(b, islot):
        pltpu.make_async_copy(
            rows_v.at[b], acc_sh.at[idx_v.at[islot, 1]], sem_s.at[b]
        ).wait()

    def iteration(i, base, retire, previdx, do_scatter):
        if retire:
            wait_scatter(i % R, (i + R) % NI)
        if previdx:
            start_idx((i + R) % NI, base + i + R)
        wait_idx(i % NI)
        start_gather(i % R, i % NI)
        if do_scatter:
            wait_gather((i - K) % R, (i - K) % NI)
            start_scatter((i - K) % R, (i - K) % NI)

    for c in range(NI):
        start_idx(c, c)
    plsc.subcore_barrier()

    for i in range(2 * R):
        iteration(i, 0, retire=(i >= R), previdx=(i >= R), do_scatter=(i >= K))

    n_groups = (M - 2 * R) // (2 * R)

    def group(g, carry):
        base = 2 * R + g * 2 * R
        for r in range(2 * R):
            iteration(r, base, retire=True, previdx=True, do_scatter=True)
        return carry

    lax.fori_loop(0, n_groups - 1, group, 0, unroll=False)

    base = M - 2 * R
    for r in range(2 * R):
        iteration(r, base, retire=True, previdx=(r < R), do_scatter=True)

    for j in range(K):
        i = M + j
        wait_gather((i - K) % R, (i - K) % NI)
        start_scatter((i - K) % R, (i - K) % NI)
    for j in range(R):
        wait_scatter((M - R + j) % R, (M - R + j) % NI)

    plsc.subcore_barrier()
    pltpu.sync_copy(
        acc_sh.at[pl.ds(sid * RPS, RPS)],
        out_hbm.at[cid, pl.ds(sid * RPS, RPS)],
    )


def _make_conv_kernel(d, r, ch):
    return functools.partial(
        pl.kernel,
        out_type=jax.ShapeDtypeStruct((NC, N_PAD, d), jnp.float32),
        mesh=_mesh,
        compiler_params=pltpu.CompilerParams(use_tc_tiling_on_sc=(d == 128)),
        scratch_types=[
            pltpu.VMEM((2 * r, 2, ch), jnp.int32),
            pltpu.VMEM((r, ch, d), jnp.float32),
            pltpu.VMEM_SHARED((N_PAD, d), jnp.float32),
            pltpu.SemaphoreType.DMA((2 * r,)),
            pltpu.SemaphoreType.DMA((r,)),
            pltpu.SemaphoreType.DMA((r,)),
        ],
    )(functools.partial(_conv_body, r, ch))


CH128 = 128
CH64 = 128
_conv128 = _make_conv_kernel(D_HID, 2, CH128)
_conv64 = _make_conv_kernel(D_OUT, 2, CH64)


BR = 1280
GRID = N_PAD // BR


def _dm(pid, deg0, deg1):
    rows = lax.broadcasted_iota(jnp.int32, (BR, 1), 0) + pid * BR
    deg = deg0[...] + deg1[...] + 1.0
    return jnp.where(rows < N_NODES, lax.rsqrt(deg), 0.0)


def _mm1_body(x_ref, w_ref, deg0, deg1, o_ref):
    dm = _dm(pl.program_id(0), deg0, deg1)
    o_ref[...] = jnp.dot(
        x_ref[...], w_ref[...], preferred_element_type=jnp.float32
    ) * dm


def _mid_body(a0, a1, hp, deg0, deg1, b1, w2, o_ref):
    dm = _dm(pl.program_id(0), deg0, deg1)
    z1 = jnp.maximum((a0[...] + a1[...] + hp[...]) * dm + b1[...], 0.0)
    o_ref[...] = jnp.dot(z1, w2[...], preferred_element_type=jnp.float32) * dm


def _fin_body(a0, a1, hp, deg0, deg1, b2, wd1, bd1, wd2, bd2, o_ref):
    dm = _dm(pl.program_id(0), deg0, deg1)
    z2 = jnp.maximum((a0[...] + a1[...] + hp[...]) * dm + b2[...], 0.0)
    z3 = jnp.maximum(
        jnp.dot(z2, wd1[...], preferred_element_type=jnp.float32) + bd1[...], 0.0
    )
    o_ref[...] = (
        jnp.dot(z3, wd2[...], preferred_element_type=jnp.float32) + bd2[...]
    )


def _row_spec(d):
    return pl.BlockSpec((BR, d), lambda i: (i, 0))


def _full_spec(r, c):
    return pl.BlockSpec((r, c), lambda i: (0, 0))


_deg_spec = pl.BlockSpec((BR, 1), lambda i: (i, 0))


def _mm1(xp, W1, deg0, deg1):
    return pl.pallas_call(
        _mm1_body,
        grid=(GRID,),
        in_specs=[_row_spec(D_IN), _full_spec(D_IN, D_HID), _deg_spec, _deg_spec],
        out_specs=_row_spec(D_HID),
        out_shape=jax.ShapeDtypeStruct((N_PAD, D_HID), jnp.float32),
    )(xp, W1, deg0, deg1)


def _mid(a0, a1, hp, deg0, deg1, b1, W2):
    return pl.pallas_call(
        _mid_body,
        grid=(GRID,),
        in_specs=[
            _row_spec(D_HID), _row_spec(D_HID), _row_spec(D_HID),
            _deg_spec, _deg_spec,
            _full_spec(1, D_HID), _full_spec(D_HID, D_OUT),
        ],
        out_specs=_row_spec(D_OUT),
        out_shape=jax.ShapeDtypeStruct((N_PAD, D_OUT), jnp.float32),
    )(a0, a1, hp, deg0, deg1, b1, W2)


def _fin(a0, a1, hp, deg0, deg1, b2, Wd1, bd1, Wd2, bd2):
    return pl.pallas_call(
        _fin_body,
        grid=(GRID,),
        in_specs=[
            _row_spec(D_OUT), _row_spec(D_OUT), _row_spec(D_OUT),
            _deg_spec, _deg_spec,
            _full_spec(1, D_OUT), _full_spec(D_OUT, D_HID),
            _full_spec(1, D_HID), _full_spec(D_HID, D_IN), _full_spec(1, D_IN),
        ],
        out_specs=_row_spec(D_IN),
        out_shape=jax.ShapeDtypeStruct((N_PAD, D_IN), jnp.float32),
    )(a0, a1, hp, deg0, deg1, b2, Wd1, bd1, Wd2, bd2)


def kernel(x, edge_index, W1, b1, W2, b2, Wd1, bd1, Wd2, bd2):
    src = edge_index[0].astype(jnp.int32)
    dst = edge_index[1].astype(jnp.int32)
    ew_real = N_EDGES // NW
    pad_w = EW - ew_real
    pad_src = jnp.full((NW, pad_w), PAD_SRC, jnp.int32)
    pad_dst = jnp.broadcast_to(
        PAD_DST + (jnp.arange(pad_w, dtype=jnp.int32) % (N_PAD - PAD_DST)),
        (NW, pad_w),
    )
    srcp = jnp.concatenate([src.reshape(NW, ew_real), pad_src], axis=1).reshape(-1)
    dstp = jnp.concatenate([dst.reshape(NW, ew_real), pad_dst], axis=1).reshape(-1)
    sdA = jnp.stack(
        [srcp.reshape(-1, CH128), dstp.reshape(-1, CH128)], axis=1
    )
    sdB = jnp.stack(
        [srcp.reshape(-1, CH64), dstp.reshape(-1, CH64)], axis=1
    )

    xp = jnp.zeros((N_PAD, D_IN), jnp.float32).at[:N_NODES].set(x)
    zeros1 = jnp.zeros((N_PAD,), jnp.float32)
    zerosA = jnp.zeros((N_PAD, D_HID), jnp.float32)
    zerosB = jnp.zeros((N_PAD, D_OUT), jnp.float32)

    degp = _deg_kernel(dstp, zeros1)
    deg0 = degp[0][:, None]
    deg1 = degp[1][:, None]

    h1p = _mm1(xp, W1, deg0, deg1)
    acc1 = _conv128(h1p, sdA, zerosA)
    h2p = _mid(acc1[0], acc1[1], h1p, deg0, deg1,
               b1[None, :], W2)
    acc2 = _conv64(h2p, sdB, zerosB)
    xh = _fin(acc2[0], acc2[1], h2p, deg0, deg1,
              b2[None, :], Wd1, bd1[None, :], Wd2, bd2[None, :])
    return xh[:N_NODES]


# speedup vs baseline: 1.9341x; 1.9341x over previous
import functools

import jax
import jax.numpy as jnp
from jax import lax
from jax.experimental import pallas as pl
from jax.experimental.pallas import tpu as pltpu
from jax.experimental.pallas import tpu_sc as plsc

N_NODES = 10000
N_EDGES = 320000
D_IN = 128
D_HID = 128
D_OUT = 64

NC = 2
NS = 16
NW = NC * NS

N_PAD = 10240
E_PAD = 327680
EW = E_PAD // NW
CHUNK = 128
NCHUNK = EW // CHUNK
RPS = N_PAD // NS

PAD_SRC = 10000
PAD_DST = 10016

_mesh = plsc.VectorSubcoreMesh(
    core_axis_name="c", subcore_axis_name="s", num_cores=NC, num_subcores=NS
)


def _deg_body(dst_hbm, zeros_hbm, out_hbm, dst_v, ones_v, acc_sh):
    cid = lax.axis_index("c")
    sid = lax.axis_index("s")
    wid = cid * NS + sid
    pltpu.sync_copy(
        zeros_hbm.at[pl.ds(sid * RPS, RPS)], acc_sh.at[pl.ds(sid * RPS, RPS)]
    )
    for j in range(CHUNK // 16):
        ones_v[pl.ds(j * 16, 16)] = jnp.full((16,), 1.0, jnp.float32)
    plsc.subcore_barrier()

    base = wid * EW

    def step(i, carry):
        off = base + i * CHUNK
        pltpu.sync_copy(dst_hbm.at[pl.ds(off, CHUNK)], dst_v)
        pltpu.sync_copy(ones_v, acc_sh.at[dst_v], add=True)
        return carry

    lax.fori_loop(0, NCHUNK, step, 0)
    plsc.subcore_barrier()
    pltpu.sync_copy(
        acc_sh.at[pl.ds(sid * RPS, RPS)], out_hbm.at[cid, pl.ds(sid * RPS, RPS)]
    )


_deg_kernel = functools.partial(
    pl.kernel,
    out_type=jax.ShapeDtypeStruct((NC, N_PAD), jnp.float32),
    mesh=_mesh,
    scratch_types=[
        pltpu.VMEM((CHUNK,), jnp.int32),
        pltpu.VMEM((CHUNK,), jnp.float32),
        pltpu.VMEM_SHARED((N_PAD,), jnp.float32),
    ],
)(_deg_body)


def _conv_body(R, ch, feat_split, h_hbm, sd_hbm, zeros_hbm, out_hbm,
               idx_v, rows_v, table_sh, acc_sh, sem_i, sem_g, sem_s):
    K = R // 2
    NI = 2 * R
    cid = lax.axis_index("c")
    sid = lax.axis_index("s")
    wid = cid * NS + sid
    pltpu.sync_copy(
        zeros_hbm.at[pl.ds(sid * RPS, RPS)], acc_sh.at[pl.ds(sid * RPS, RPS)]
    )
    if feat_split:
        M = (E_PAD // NS) // ch
        cbase = sid * M
        pltpu.sync_copy(
            h_hbm.at[pl.ds(sid * RPS, RPS), pl.ds(cid * (D_HID // 2), D_HID // 2)],
            table_sh.at[pl.ds(sid * RPS, RPS)],
        )
    else:
        M = EW // ch
        cbase = wid * M
        pltpu.sync_copy(
            h_hbm.at[pl.ds(sid * RPS, RPS)], table_sh.at[pl.ds(sid * RPS, RPS)]
        )

    def start_idx(islot, chunk):
        pltpu.async_copy(sd_hbm.at[cbase + chunk], idx_v.at[islot], sem_i.at[islot])

    def wait_idx(islot):
        pltpu.make_async_copy(
            sd_hbm.at[cbase], idx_v.at[islot], sem_i.at[islot]
        ).wait()

    def start_gather(b, islot):
        pltpu.async_copy(table_sh.at[idx_v.at[islot, 0]], rows_v.at[b], sem_g.at[b])

    def wait_gather(b, islot):
        pltpu.make_async_copy(
            table_sh.at[idx_v.at[islot, 0]], rows_v.at[b], sem_g.at[b]
        ).wait()

    def start_scatter(b, islot):
        pltpu.async_copy(
            rows_v.at[b], acc_sh.at[idx_v.at[islot, 1]], sem_s.at[b], add=True
        )

    def wait_scatter(b, islot):
        pltpu.make_async_copy(
            rows_v.at[b], acc_sh.at[idx_v.at[islot, 1]], sem_s.at[b]
        ).wait()

    def iteration(i, base, retire, previdx, do_scatter):
        if retire:
            wait_scatter(i % R, (i + R) % NI)
        if previdx:
            start_idx((i + R) % NI, base + i + R)
        wait_idx(i % NI)
        start_gather(i % R, i % NI)
        if do_scatter:
            wait_gather((i - K) % R, (i - K) % NI)
            start_scatter((i - K) % R, (i - K) % NI)

    for c in range(NI):
        start_idx(c, c)
    plsc.subcore_barrier()

    for i in range(2 * R):
        iteration(i, 0, retire=(i >= R), previdx=(i >= R), do_scatter=(i >= K))

    n_groups = (M - 2 * R) // (2 * R)

    def group(g, carry):
        base = 2 * R + g * 2 * R
        for r in range(2 * R):
            iteration(r, base, retire=True, previdx=True, do_scatter=True)
        return carry

    lax.fori_loop(0, n_groups - 1, group, 0, unroll=False)

    base = M - 2 * R
    for r in range(2 * R):
        iteration(r, base, retire=True, previdx=(r < R), do_scatter=True)

    for j in range(K):
        i = M + j
        wait_gather((i - K) % R, (i - K) % NI)
        start_scatter((i - K) % R, (i - K) % NI)
    for j in range(R):
        wait_scatter((M - R + j) % R, (M - R + j) % NI)

    plsc.subcore_barrier()
    pltpu.sync_copy(
        acc_sh.at[pl.ds(sid * RPS, RPS)],
        out_hbm.at[cid, pl.ds(sid * RPS, RPS)],
    )


def _make_conv_kernel(d, r, ch, feat_split):
    return functools.partial(
        pl.kernel,
        out_type=jax.ShapeDtypeStruct((NC, N_PAD, d), jnp.float32),
        mesh=_mesh,
        compiler_params=pltpu.CompilerParams(use_tc_tiling_on_sc=False),
        scratch_types=[
            pltpu.VMEM((2 * r, 2, ch), jnp.int32),
            pltpu.VMEM((r, ch, d), jnp.float32),
            pltpu.VMEM_SHARED((N_PAD, d), jnp.float32),
            pltpu.VMEM_SHARED((N_PAD, d), jnp.float32),
            pltpu.SemaphoreType.DMA((2 * r,)),
            pltpu.SemaphoreType.DMA((r,)),
            pltpu.SemaphoreType.DMA((r,)),
        ],
    )(functools.partial(_conv_body, r, ch, feat_split))


CH128 = 128
CH64 = 128
_conv128 = _make_conv_kernel(D_HID // 2, 2, CH128, True)
_conv64 = _make_conv_kernel(D_OUT, 2, CH64, False)


BR = 1280
GRID = N_PAD // BR


def _dm(pid, deg0, deg1):
    rows = lax.broadcasted_iota(jnp.int32, (BR, 1), 0) + pid * BR
    deg = deg0[...] + deg1[...] + 1.0
    return jnp.where(rows < N_NODES, lax.rsqrt(deg), 0.0)


def _mm1_body(x_ref, w_ref, deg0, deg1, o_ref):
    dm = _dm(pl.program_id(0), deg0, deg1)
    o_ref[...] = jnp.dot(
        x_ref[...], w_ref[...], preferred_element_type=jnp.float32
    ) * dm


def _mid_body(a0, a1, hp, deg0, deg1, b1, w2, o_ref):
    dm = _dm(pl.program_id(0), deg0, deg1)
    s = jnp.concatenate([a0[...], a1[...]], axis=1)
    z1 = jnp.maximum((s + hp[...]) * dm + b1[...], 0.0)
    o_ref[...] = jnp.dot(z1, w2[...], preferred_element_type=jnp.float32) * dm


def _fin_body(a0, a1, hp, deg0, deg1, b2, wd1, bd1, wd2, bd2, o_ref):
    dm = _dm(pl.program_id(0), deg0, deg1)
    z2 = jnp.maximum((a0[...] + a1[...] + hp[...]) * dm + b2[...], 0.0)
    z3 = jnp.maximum(
        jnp.dot(z2, wd1[...], preferred_element_type=jnp.float32) + bd1[...], 0.0
    )
    o_ref[...] = (
        jnp.dot(z3, wd2[...], preferred_element_type=jnp.float32) + bd2[...]
    )


def _row_spec(d):
    return pl.BlockSpec((BR, d), lambda i: (i, 0))


def _full_spec(r, c):
    return pl.BlockSpec((r, c), lambda i: (0, 0))


_deg_spec = pl.BlockSpec((BR, 1), lambda i: (i, 0))


def _mm1(xp, W1, deg0, deg1):
    return pl.pallas_call(
        _mm1_body,
        grid=(GRID,),
        in_specs=[_row_spec(D_IN), _full_spec(D_IN, D_HID), _deg_spec, _deg_spec],
        out_specs=_row_spec(D_HID),
        out_shape=jax.ShapeDtypeStruct((N_PAD, D_HID), jnp.float32),
    )(xp, W1, deg0, deg1)


def _mid(a0, a1, hp, deg0, deg1, b1, W2):
    return pl.pallas_call(
        _mid_body,
        grid=(GRID,),
        in_specs=[
            _row_spec(D_HID // 2), _row_spec(D_HID // 2), _row_spec(D_HID),
            _deg_spec, _deg_spec,
            _full_spec(1, D_HID), _full_spec(D_HID, D_OUT),
        ],
        out_specs=_row_spec(D_OUT),
        out_shape=jax.ShapeDtypeStruct((N_PAD, D_OUT), jnp.float32),
    )(a0, a1, hp, deg0, deg1, b1, W2)


def _fin(a0, a1, hp, deg0, deg1, b2, Wd1, bd1, Wd2, bd2):
    return pl.pallas_call(
        _fin_body,
        grid=(GRID,),
        in_specs=[
            _row_spec(D_OUT), _row_spec(D_OUT), _row_spec(D_OUT),
            _deg_spec, _deg_spec,
            _full_spec(1, D_OUT), _full_spec(D_OUT, D_HID),
            _full_spec(1, D_HID), _full_spec(D_HID, D_IN), _full_spec(1, D_IN),
        ],
        out_specs=_row_spec(D_IN),
        out_shape=jax.ShapeDtypeStruct((N_PAD, D_IN), jnp.float32),
    )(a0, a1, hp, deg0, deg1, b2, Wd1, bd1, Wd2, bd2)


def kernel(x, edge_index, W1, b1, W2, b2, Wd1, bd1, Wd2, bd2):
    src = edge_index[0].astype(jnp.int32)
    dst = edge_index[1].astype(jnp.int32)
    ew_real = N_EDGES // NW
    pad_w = EW - ew_real
    pad_src = jnp.full((NW, pad_w), PAD_SRC, jnp.int32)
    pad_dst = jnp.broadcast_to(
        PAD_DST + (jnp.arange(pad_w, dtype=jnp.int32) % (N_PAD - PAD_DST)),
        (NW, pad_w),
    )
    srcp = jnp.concatenate([src.reshape(NW, ew_real), pad_src], axis=1).reshape(-1)
    dstp = jnp.concatenate([dst.reshape(NW, ew_real), pad_dst], axis=1).reshape(-1)
    sdA = jnp.stack(
        [srcp.reshape(-1, CH128), dstp.reshape(-1, CH128)], axis=1
    )
    sdB = jnp.stack(
        [srcp.reshape(-1, CH64), dstp.reshape(-1, CH64)], axis=1
    )

    xp = jnp.zeros((N_PAD, D_IN), jnp.float32).at[:N_NODES].set(x)
    zeros1 = jnp.zeros((N_PAD,), jnp.float32)
    zerosB = jnp.zeros((N_PAD, D_OUT), jnp.float32)

    degp = _deg_kernel(dstp, zeros1)
    deg0 = degp[0][:, None]
    deg1 = degp[1][:, None]

    h1p = _mm1(xp, W1, deg0, deg1)
    acc1 = _conv128(h1p, sdA, zerosB)
    h2p = _mid(acc1[0], acc1[1], h1p, deg0, deg1,
               b1[None, :], W2)
    acc2 = _conv64(h2p, sdB, zerosB)
    xh = _fin(acc2[0], acc2[1], h2p, deg0, deg1,
              b2[None, :], Wd1, bd1[None, :], Wd2, bd2[None, :])
    return xh[:N_NODES]


# speedup vs baseline: 2.2320x; 1.1540x over previous
import functools

import jax
import jax.numpy as jnp
from jax import lax
from jax.experimental import pallas as pl
from jax.experimental.pallas import tpu as pltpu
from jax.experimental.pallas import tpu_sc as plsc

N_NODES = 10000
N_EDGES = 320000
D_IN = 128
D_HID = 128
D_OUT = 64

NC = 2
NS = 16
NW = NC * NS

N_PAD = 10240
E_PAD = 327680
EW = E_PAD // NW
CH = 128
RPS = N_PAD // NS

PAD_SRC = 10000
PAD_DST = 10016

M64 = EW // CH
M128 = (E_PAD // NS) // CH

_mesh = plsc.VectorSubcoreMesh(
    core_axis_name="c", subcore_axis_name="s", num_cores=NC, num_subcores=NS
)


def _deg_body(sd_hbm, zeros_hbm, out_hbm, idx_all, ones_v, acc_sh, sem_s):
    cid = lax.axis_index("c")
    sid = lax.axis_index("s")
    wid = cid * NS + sid
    pltpu.sync_copy(
        zeros_hbm.at[pl.ds(sid * RPS, RPS)], acc_sh.at[pl.ds(sid * RPS, RPS)]
    )
    pltpu.sync_copy(sd_hbm.at[wid], idx_all)
    for j in range(CH // 16):
        ones_v[pl.ds(j * 16, 16)] = jnp.full((16,), 1.0, jnp.float32)
    plsc.subcore_barrier()

    def start_scatter(b, c):
        pltpu.async_copy(ones_v, acc_sh.at[idx_all.at[c]], sem_s.at[b], add=True)

    def wait_scatter(b, c):
        pltpu.make_async_copy(
            ones_v, acc_sh.at[idx_all.at[c]], sem_s.at[b]
        ).wait()

    start_scatter(0, 0)
    start_scatter(1, 1)

    def group(g, carry):
        base = 2 + g * 2
        for r in range(2):
            wait_scatter(r, base + r - 2)
            start_scatter(r, base + r)
        return carry

    lax.fori_loop(0, (M64 - 2) // 2, group, 0, unroll=False)
    wait_scatter(0, M64 - 2)
    wait_scatter(1, M64 - 1)

    plsc.subcore_barrier()
    pltpu.sync_copy(
        acc_sh.at[pl.ds(sid * RPS, RPS)], out_hbm.at[cid, pl.ds(sid * RPS, RPS)]
    )


_deg_kernel = functools.partial(
    pl.kernel,
    out_type=jax.ShapeDtypeStruct((NC, N_PAD), jnp.float32),
    mesh=_mesh,
    scratch_types=[
        pltpu.VMEM((M64, CH), jnp.int32),
        pltpu.VMEM((CH,), jnp.float32),
        pltpu.VMEM_SHARED((N_PAD,), jnp.float32),
        pltpu.SemaphoreType.DMA((2,)),
    ],
)(_deg_body)


def _conv_body(R, M, d, feat_split, spans, h_hbm, sd_hbm, zeros_hbm, out_hbm,
               idx_all, rows_v, table_sh, acc_sh, sem_g, sem_s):
    K = R // 2
    cid = lax.axis_index("c")
    sid = lax.axis_index("s")
    wid = cid * NS + sid
    pltpu.sync_copy(
        zeros_hbm.at[pl.ds(sid * RPS, RPS)], acc_sh.at[pl.ds(sid * RPS, RPS)]
    )
    if feat_split:
        widx = sid
        pltpu.sync_copy(
            h_hbm.at[pl.ds(sid * RPS, RPS), pl.ds(cid * d, d)],
            table_sh.at[pl.ds(sid * RPS, RPS)],
        )
    else:
        widx = wid
        pltpu.sync_copy(
            h_hbm.at[pl.ds(sid * RPS, RPS)], table_sh.at[pl.ds(sid * RPS, RPS)]
        )
    H = M // spans

    def start_gather(b, c):
        pltpu.async_copy(table_sh.at[idx_all.at[0, c]], rows_v.at[b], sem_g.at[b])

    def wait_gather(b, c):
        pltpu.make_async_copy(
            table_sh.at[idx_all.at[0, c]], rows_v.at[b], sem_g.at[b]
        ).wait()

    def start_scatter(b, c):
        pltpu.async_copy(
            rows_v.at[b], acc_sh.at[idx_all.at[1, c]], sem_s.at[b], add=True
        )

    def wait_scatter(b, c):
        pltpu.make_async_copy(
            rows_v.at[b], acc_sh.at[idx_all.at[1, c]], sem_s.at[b]
        ).wait()

    plsc.subcore_barrier()

    def group(g, carry):
        base = R + g * R
        for r in range(R):
            c = base + r
            wait_scatter(r, c - R)
            start_gather(r, c)
            wait_gather((r - K) % R, c - K)
            start_scatter((r - K) % R, c - K)
        return carry

    for s in range(spans):
        pltpu.sync_copy(sd_hbm.at[widx, s], idx_all)
        for i in range(R):
            start_gather(i, i)
            if i >= K:
                wait_gather(i - K, i - K)
                start_scatter(i - K, i - K)
        lax.fori_loop(0, H // R - 1, group, 0, unroll=False)
        for j in range(K):
            c = H - K + j
            wait_gather(c % R, c)
            start_scatter(c % R, c)
        for j in range(R):
            c = H - R + j
            wait_scatter(c % R, c)

    plsc.subcore_barrier()
    pltpu.sync_copy(
        acc_sh.at[pl.ds(sid * RPS, RPS)],
        out_hbm.at[cid, pl.ds(sid * RPS, RPS)],
    )


def _make_conv_kernel(d, r, m, feat_split, spans):
    return functools.partial(
        pl.kernel,
        out_type=jax.ShapeDtypeStruct((NC, N_PAD, d), jnp.float32),
        mesh=_mesh,
        compiler_params=pltpu.CompilerParams(use_tc_tiling_on_sc=False),
        scratch_types=[
            pltpu.VMEM((2, m // spans, CH), jnp.int32),
            pltpu.VMEM((r, CH, d), jnp.float32),
            pltpu.VMEM_SHARED((N_PAD, d), jnp.float32),
            pltpu.VMEM_SHARED((N_PAD, d), jnp.float32),
            pltpu.SemaphoreType.DMA((r,)),
            pltpu.SemaphoreType.DMA((r,)),
        ],
    )(functools.partial(_conv_body, r, m, d, feat_split, spans))


_conv128 = _make_conv_kernel(D_HID // 2, 2, M128, True, 2)
_conv64 = _make_conv_kernel(D_OUT, 2, M64, False, 1)


BR = 1280
GRID = N_PAD // BR


def _dm(pid, deg_ref):
    rows = lax.broadcasted_iota(jnp.int32, (BR, 1), 0) + pid * BR
    deg = deg_ref[0] + deg_ref[1] + 1.0
    return jnp.where(rows < N_NODES, lax.rsqrt(deg), 0.0)


def _mm1_body(x_ref, w_ref, deg_ref, o_ref):
    dm = _dm(pl.program_id(0), deg_ref)
    o_ref[...] = jnp.dot(
        x_ref[...], w_ref[...], preferred_element_type=jnp.float32
    ) * dm


def _mid_body(acc_ref, hp, deg_ref, b1, w2, o_ref):
    dm = _dm(pl.program_id(0), deg_ref)
    s = jnp.concatenate([acc_ref[0], acc_ref[1]], axis=1)
    z1 = jnp.maximum((s + hp[...]) * dm + b1[...], 0.0)
    o_ref[...] = jnp.dot(z1, w2[...], preferred_element_type=jnp.float32) * dm


def _fin_body(acc_ref, hp, deg_ref, b2, wd1, bd1, wd2, bd2, o_ref):
    dm = _dm(pl.program_id(0), deg_ref)
    z2 = jnp.maximum((acc_ref[0] + acc_ref[1] + hp[...]) * dm + b2[...], 0.0)
    z3 = jnp.maximum(
        jnp.dot(z2, wd1[...], preferred_element_type=jnp.float32) + bd1[...], 0.0
    )
    o_ref[...] = (
        jnp.dot(z3, wd2[...], preferred_element_type=jnp.float32) + bd2[...]
    )


def _row_spec(d):
    return pl.BlockSpec((BR, d), lambda i: (i, 0))


def _full_spec(r, c):
    return pl.BlockSpec((r, c), lambda i: (0, 0))


_deg_spec = pl.BlockSpec((2, BR, 1), lambda i: (0, i, 0))


def _acc_spec(d):
    return pl.BlockSpec((2, BR, d), lambda i: (0, i, 0))


def _mm1(xp, W1, deg3):
    return pl.pallas_call(
        _mm1_body,
        grid=(GRID,),
        in_specs=[_row_spec(D_IN), _full_spec(D_IN, D_HID), _deg_spec],
        out_specs=_row_spec(D_HID),
        out_shape=jax.ShapeDtypeStruct((N_PAD, D_HID), jnp.float32),
    )(xp, W1, deg3)


def _mid(acc1, hp, deg3, b1, W2):
    return pl.pallas_call(
        _mid_body,
        grid=(GRID,),
        in_specs=[
            _acc_spec(D_HID // 2), _row_spec(D_HID), _deg_spec,
            _full_spec(1, D_HID), _full_spec(D_HID, D_OUT),
        ],
        out_specs=_row_spec(D_OUT),
        out_shape=jax.ShapeDtypeStruct((N_PAD, D_OUT), jnp.float32),
    )(acc1, hp, deg3, b1, W2)


def _fin(acc2, hp, deg3, b2, Wd1, bd1, Wd2, bd2):
    return pl.pallas_call(
        _fin_body,
        grid=(GRID,),
        in_specs=[
            _acc_spec(D_OUT), _row_spec(D_OUT), _deg_spec,
            _full_spec(1, D_OUT), _full_spec(D_OUT, D_HID),
            _full_spec(1, D_HID), _full_spec(D_HID, D_IN), _full_spec(1, D_IN),
        ],
        out_specs=_row_spec(D_IN),
        out_shape=jax.ShapeDtypeStruct((N_PAD, D_IN), jnp.float32),
    )(acc2, hp, deg3, b2, Wd1, bd1, Wd2, bd2)


def kernel(x, edge_index, W1, b1, W2, b2, Wd1, bd1, Wd2, bd2):
    src = edge_index[0].astype(jnp.int32)
    dst = edge_index[1].astype(jnp.int32)
    ew_real = N_EDGES // NW
    pad_w = EW - ew_real
    pad_src = jnp.full((NW, pad_w), PAD_SRC, jnp.int32)
    pad_dst = jnp.broadcast_to(
        PAD_DST + (jnp.arange(pad_w, dtype=jnp.int32) % (N_PAD - PAD_DST)),
        (NW, pad_w),
    )
    srcp = jnp.concatenate([src.reshape(NW, ew_real), pad_src], axis=1).reshape(-1)
    dstp = jnp.concatenate([dst.reshape(NW, ew_real), pad_dst], axis=1).reshape(-1)
    sd128 = jnp.stack(
        [srcp.reshape(NS, 2, M128 // 2, CH), dstp.reshape(NS, 2, M128 // 2, CH)],
        axis=2,
    )
    sd64 = jnp.stack(
        [srcp.reshape(NW, 1, M64, CH), dstp.reshape(NW, 1, M64, CH)], axis=2
    )
    sd_deg = dstp.reshape(NW, M64, CH)

    xp = jnp.zeros((N_PAD, D_IN), jnp.float32).at[:N_NODES].set(x)
    zeros1 = jnp.zeros((N_PAD,), jnp.float32)
    zerosB = jnp.zeros((N_PAD, D_OUT), jnp.float32)

    degp = _deg_kernel(sd_deg, zeros1)
    deg3 = degp[:, :, None]

    h1p = _mm1(xp, W1, deg3)
    acc1 = _conv128(h1p, sd128, zerosB)
    h2p = _mid(acc1, h1p, deg3, b1[None, :], W2)
    acc2 = _conv64(h2p, sd64, zerosB)
    xh = _fin(acc2, h2p, deg3, b2[None, :],
              Wd1, bd1[None, :], Wd2, bd2[None, :])
    return xh[:N_NODES]
